# baseline (speedup 1.0000x reference)
import functools

import jax
import jax.numpy as jnp
from jax import lax
from jax.experimental import pallas as pl
from jax.experimental.pallas import tpu as pltpu
from jax.experimental.pallas import tpu_sc as plsc

N_TABLES = 26
NUM_EMB = 100000
EMB_DIM = 32

_NC, _NS = 2, 16
_NW = _NC * _NS
_R = 64
_KK = _R * N_TABLES
_GROUPS = _KK // 16
_LANE = 16


def _wide_embed(x_flat, w2, rep, off, *, total):
    bc = total // _NW
    rounds = bc // (2 * _R)

    mesh = plsc.VectorSubcoreMesh(core_axis_name="c", subcore_axis_name="s")

    @functools.partial(
        pl.kernel,
        mesh=mesh,
        out_type=jax.ShapeDtypeStruct((total * N_TABLES, EMB_DIM), jnp.float32),
        scratch_types=[
            pltpu.VMEM((bc,), jnp.int32),
            pltpu.VMEM((_KK,), jnp.int32),
            pltpu.VMEM((_KK,), jnp.int32),
            pltpu.VMEM((2, _KK), jnp.int32),
            pltpu.VMEM((2, _KK, EMB_DIM), jnp.float32),
            pltpu.SemaphoreType.DMA,
            pltpu.SemaphoreType.DMA,
            pltpu.SemaphoreType.DMA,
            pltpu.SemaphoreType.DMA,
        ],
        compiler_params=pltpu.CompilerParams(
            use_tc_tiling_on_sc=False, needs_layout_passes=False
        ),
    )
    def k(
        w_hbm, idx_hbm, rep_hbm, off_hbm, out_hbm,
        idx_v, rep_v, off_v, idx2_v, rows_v, g0, g1, w0, w1,
    ):
        wid = lax.axis_index("s") * _NC + lax.axis_index("c")
        base = wid * bc
        pltpu.sync_copy(idx_hbm.at[pl.ds(base, bc)], idx_v)
        pltpu.sync_copy(rep_hbm, rep_v)
        pltpu.sync_copy(off_hbm, off_v)

        gsems = (g0, g1)
        wsems = (w0, w1)

        def build(j, c):
            rowbase = c * _R

            @pl.loop(0, _GROUPS)
            def _(gi):
                sl = pl.ds(gi * _LANE, _LANE)
                row = rowbase + rep_v[sl]
                xg = plsc.load_gather(idx_v, [row])
                idx2_v[j, sl] = xg + off_v[sl]

        def gather(j):
            pltpu.async_copy(w_hbm.at[idx2_v.at[j]], rows_v.at[j], gsems[j])

        def gather_wait(j):
            pltpu.make_async_copy(
                w_hbm.at[idx2_v.at[j]], rows_v.at[j], gsems[j]
            ).wait()

        def wb(j, c):
            return pltpu.make_async_copy(
                rows_v.at[j],
                out_hbm.at[pl.ds((base + c * _R) * N_TABLES, _KK)],
                wsems[j],
            )

        @pl.loop(0, rounds)
        def round_loop(r):
            for j in range(2):
                @pl.when(r > 0)
                def _(j=j):
                    wb(j, (r - 1) * 2 + j).wait()

                build(j, r * 2 + j)
                gather(j)
            for j in range(2):
                gather_wait(j)
                wb(j, r * 2 + j).start()

        for j in range(2):
            wb(j, (rounds - 1) * 2 + j).wait()

    return k(w2, x_flat, rep, off)


def kernel(x, weight):
    B, T = x.shape
    total = B * T
    w2 = weight.reshape(N_TABLES * NUM_EMB, EMB_DIM)
    karr = jnp.arange(_KK, dtype=jnp.int32)
    rep = karr // N_TABLES
    off = (karr % N_TABLES) * NUM_EMB
    out = _wide_embed(x.reshape(total), w2, rep, off, total=total)
    return out.reshape(B, T, N_TABLES * EMB_DIM)
